# baseline (speedup 1.0000x reference)
import functools

import jax
import jax.numpy as jnp
from jax import lax
from jax.experimental import pallas as pl
from jax.experimental.pallas import tpu as pltpu
from jax.experimental.pallas import tpu_sc as plsc

_IN_DIM = 100000
_OUT_DIM = 64
_PACK = 2 * _OUT_DIM
_BATCH = 16384

_TR_COLS = 8192
_TR_GRID = 7
_HALF = _TR_COLS * _TR_GRID
_TR_LAST_SAFE = -(-_IN_DIM // _TR_COLS) - 1


def _transpose_body(a_ref, b_ref, eye_ref, o_ref):
    s = jnp.concatenate([a_ref[...], b_ref[...]], axis=0)
    t = lax.dot_general(
        s, eye_ref[...], (((0,), (0,)), ((), ())),
        preferred_element_type=jnp.float32,
    )
    o_ref[...] = pltpu.bitcast(t.astype(jnp.bfloat16), jnp.uint32)


def _transpose_packed(weights):
    return pl.pallas_call(
        _transpose_body,
        grid=(_TR_GRID,),
        in_specs=[
            pl.BlockSpec((_OUT_DIM, _TR_COLS), lambda i: (0, i)),
            pl.BlockSpec(
                (_OUT_DIM, _TR_COLS),
                lambda i: (0, jnp.minimum(i + _TR_GRID, _TR_LAST_SAFE)),
            ),
            pl.BlockSpec((_PACK, _PACK), lambda i: (0, 0)),
        ],
        out_specs=pl.BlockSpec((_TR_COLS // 2, _PACK), lambda i: (i, 0)),
        out_shape=jax.ShapeDtypeStruct((_HALF // 2, _PACK), jnp.uint32),
    )(weights, weights, jnp.eye(_PACK, dtype=jnp.float32))


def _make_gather():
    info = plsc.get_sparse_core_info()
    nc, ns = info.num_cores, info.num_subcores
    nw = nc * ns
    b_per_w = _BATCH // nw
    chunks = b_per_w // 128
    mesh = plsc.VectorSubcoreMesh(core_axis_name="c", subcore_axis_name="s")

    @functools.partial(
        pl.kernel,
        mesh=mesh,
        out_type=jax.ShapeDtypeStruct((_BATCH, _PACK), jnp.uint32),
        scratch_types=[
            pltpu.VMEM((chunks, 128), jnp.int32),
            pltpu.VMEM((b_per_w, _PACK), jnp.uint32),
            pltpu.SemaphoreType.DMA,
        ],
    )
    def gather(table_hbm, idx_hbm, out_hbm, idx_v, rows_v, sem):
        wid = lax.axis_index("s") * nc + lax.axis_index("c")
        pltpu.sync_copy(idx_hbm.at[pl.ds(wid * chunks, chunks)], idx_v)
        handles = [
            pltpu.async_copy(
                table_hbm.at[idx_v.at[k]],
                rows_v.at[pl.ds(k * 128, 128)],
                sem,
            )
            for k in range(chunks)
        ]
        for h in handles:
            h.wait()
        pltpu.sync_copy(rows_v, out_hbm.at[pl.ds(wid * b_per_w, b_per_w)])

    return gather


_SEL_ROWS = 2048


def _select_body(g_ref, p_ref, o_ref):
    sel = p_ref[...].reshape(_SEL_ROWS, 1)
    g = g_ref[...]
    w = jnp.where(sel >= 2, g[:, _OUT_DIM:], g[:, :_OUT_DIM])
    bits = jnp.where((sel & 1) != 0, w & jnp.uint32(0xFFFF0000), w << 16)
    o_ref[...] = lax.bitcast_convert_type(bits, jnp.float32)


def _select(gathered, par):
    return pl.pallas_call(
        _select_body,
        grid=(_BATCH // _SEL_ROWS,),
        in_specs=[
            pl.BlockSpec((_SEL_ROWS, _PACK), lambda i: (i, 0)),
            pl.BlockSpec((_SEL_ROWS,), lambda i: (i,)),
        ],
        out_specs=pl.BlockSpec((_SEL_ROWS, _OUT_DIM), lambda i: (i, 0)),
        out_shape=jax.ShapeDtypeStruct((_BATCH, _OUT_DIM), jnp.float32),
    )(gathered, par)


def kernel(index, weights):
    table = _transpose_packed(weights)
    idx = index.reshape(-1).astype(jnp.int32)
    h = (idx >= _HALF).astype(jnp.int32)
    r = idx - h * _HALF
    idx2 = (r >> 1).reshape(_BATCH // 128, 128)
    sel = h * 2 + (r & 1)
    gathered = _make_gather()(table, idx2)
    return _select(gathered, sel)


# speedup vs baseline: 1.0263x; 1.0263x over previous
import functools

import jax
import jax.numpy as jnp
from jax import lax
from jax.experimental import pallas as pl
from jax.experimental.pallas import tpu as pltpu
from jax.experimental.pallas import tpu_sc as plsc

_IN_DIM = 100000
_OUT_DIM = 64
_PACK = 2 * _OUT_DIM
_BATCH = 16384

_TR_COLS = 16384
_TR_GRID = 4
_HALF = _TR_COLS * _TR_GRID
_TR_LAST_SAFE = -(-_IN_DIM // _TR_COLS) - 1


def _transpose_body(a_ref, b_ref, eye_ref, o_ref):
    s = jnp.concatenate([a_ref[...], b_ref[...]], axis=0)
    t = lax.dot_general(
        s, eye_ref[...], (((0,), (0,)), ((), ())),
        preferred_element_type=jnp.float32,
    )
    o_ref[...] = pltpu.bitcast(t.astype(jnp.bfloat16), jnp.uint32)


def _transpose_packed(weights):
    return pl.pallas_call(
        _transpose_body,
        grid=(_TR_GRID,),
        in_specs=[
            pl.BlockSpec((_OUT_DIM, _TR_COLS), lambda i: (0, i)),
            pl.BlockSpec(
                (_OUT_DIM, _TR_COLS),
                lambda i: (0, jnp.minimum(i + _TR_GRID, _TR_LAST_SAFE)),
            ),
            pl.BlockSpec((_PACK, _PACK), lambda i: (0, 0)),
        ],
        out_specs=pl.BlockSpec((_TR_COLS // 2, _PACK), lambda i: (i, 0)),
        out_shape=jax.ShapeDtypeStruct((_HALF // 2, _PACK), jnp.uint32),
    )(weights, weights, jnp.eye(_PACK, dtype=jnp.float32))


def _make_gather():
    info = plsc.get_sparse_core_info()
    nc, ns = info.num_cores, info.num_subcores
    nw = nc * ns
    b_per_w = _BATCH // nw
    chunks = b_per_w // 128
    mesh = plsc.VectorSubcoreMesh(core_axis_name="c", subcore_axis_name="s")

    @functools.partial(
        pl.kernel,
        mesh=mesh,
        out_type=jax.ShapeDtypeStruct((_BATCH, _PACK), jnp.uint32),
        scratch_types=[
            pltpu.VMEM((chunks, 128), jnp.int32),
            pltpu.VMEM((b_per_w, _PACK), jnp.uint32),
            pltpu.SemaphoreType.DMA,
        ],
    )
    def gather(table_hbm, idx_hbm, out_hbm, idx_v, rows_v, sem):
        wid = lax.axis_index("s") * nc + lax.axis_index("c")
        pltpu.sync_copy(idx_hbm.at[pl.ds(wid * chunks, chunks)], idx_v)
        handles = [
            pltpu.async_copy(
                table_hbm.at[idx_v.at[k]],
                rows_v.at[pl.ds(k * 128, 128)],
                sem,
            )
            for k in range(chunks)
        ]
        for h in handles:
            h.wait()
        pltpu.sync_copy(rows_v, out_hbm.at[pl.ds(wid * b_per_w, b_per_w)])

    return gather


_SEL_ROWS = 2048


def _select_body(g_ref, p_ref, o_ref):
    sel = p_ref[...].reshape(_SEL_ROWS, 1)
    g = g_ref[...]
    w = jnp.where(sel >= 2, g[:, _OUT_DIM:], g[:, :_OUT_DIM])
    bits = jnp.where((sel & 1) != 0, w & jnp.uint32(0xFFFF0000), w << 16)
    o_ref[...] = lax.bitcast_convert_type(bits, jnp.float32)


def _select(gathered, par):
    return pl.pallas_call(
        _select_body,
        grid=(_BATCH // _SEL_ROWS,),
        in_specs=[
            pl.BlockSpec((_SEL_ROWS, _PACK), lambda i: (i, 0)),
            pl.BlockSpec((_SEL_ROWS,), lambda i: (i,)),
        ],
        out_specs=pl.BlockSpec((_SEL_ROWS, _OUT_DIM), lambda i: (i, 0)),
        out_shape=jax.ShapeDtypeStruct((_BATCH, _OUT_DIM), jnp.float32),
    )(gathered, par)


def kernel(index, weights):
    table = _transpose_packed(weights)
    idx = index.reshape(-1).astype(jnp.int32)
    h = (idx >= _HALF).astype(jnp.int32)
    r = idx - h * _HALF
    idx2 = (r >> 1).reshape(_BATCH // 128, 128)
    sel = h * 2 + (r & 1)
    gathered = _make_gather()(table, idx2)
    return _select(gathered, sel)


# speedup vs baseline: 1.0563x; 1.0292x over previous
import functools

import jax
import jax.numpy as jnp
from jax import lax
from jax.experimental import pallas as pl
from jax.experimental.pallas import tpu as pltpu
from jax.experimental.pallas import tpu_sc as plsc

_IN_DIM = 100000
_OUT_DIM = 64
_PACK = 2 * _OUT_DIM
_BATCH = 16384

_TR_COLS = 16384
_TR_GRID = 4
_HALF = _TR_COLS * _TR_GRID
_TR_LAST_SAFE = -(-_IN_DIM // _TR_COLS) - 1


def _transpose_body(a_ref, b_ref, eye_ref, o_ref):
    s = jnp.concatenate([a_ref[...], b_ref[...]], axis=0)
    t = lax.dot_general(
        s, eye_ref[...], (((0,), (0,)), ((), ())),
        preferred_element_type=jnp.float32,
    )
    o_ref[...] = pltpu.bitcast(t.astype(jnp.bfloat16), jnp.uint32)


def _transpose_packed(weights):
    return pl.pallas_call(
        _transpose_body,
        grid=(_TR_GRID,),
        in_specs=[
            pl.BlockSpec((_OUT_DIM, _TR_COLS), lambda i: (0, i)),
            pl.BlockSpec(
                (_OUT_DIM, _TR_COLS),
                lambda i: (0, jnp.minimum(i + _TR_GRID, _TR_LAST_SAFE)),
            ),
            pl.BlockSpec((_PACK, _PACK), lambda i: (0, 0)),
        ],
        out_specs=pl.BlockSpec((_TR_COLS // 2, _PACK), lambda i: (i, 0)),
        out_shape=jax.ShapeDtypeStruct((_HALF // 2, _PACK), jnp.uint32),
    )(weights, weights, jnp.eye(_PACK, dtype=jnp.float32))


def _make_gather():
    info = plsc.get_sparse_core_info()
    nc, ns = info.num_cores, info.num_subcores
    nw = nc * ns
    b_per_w = _BATCH // nw
    chunks = b_per_w // 128
    mesh = plsc.VectorSubcoreMesh(core_axis_name="c", subcore_axis_name="s")

    @functools.partial(
        pl.kernel,
        mesh=mesh,
        out_type=jax.ShapeDtypeStruct((_BATCH, _PACK), jnp.uint32),
        scratch_types=[
            pltpu.VMEM((chunks, 128), jnp.int32),
            pltpu.VMEM((b_per_w, _PACK), jnp.uint32),
            pltpu.SemaphoreType.DMA,
        ],
    )
    def gather(table_hbm, idx_hbm, out_hbm, idx_v, rows_v, sem):
        wid = lax.axis_index("s") * nc + lax.axis_index("c")
        pltpu.sync_copy(idx_hbm.at[pl.ds(wid * chunks, chunks)], idx_v)
        handles = [
            pltpu.async_copy(
                table_hbm.at[idx_v.at[k]],
                rows_v.at[pl.ds(k * 128, 128)],
                sem,
            )
            for k in range(chunks)
        ]
        for h in handles:
            h.wait()
        pltpu.sync_copy(rows_v, out_hbm.at[pl.ds(wid * b_per_w, b_per_w)])

    return gather


_SEL_ROWS = 4096


def _select_body(g_ref, p_ref, o_ref):
    sel = p_ref[...].reshape(_SEL_ROWS, 1)
    g = g_ref[...]
    w = jnp.where(sel >= 2, g[:, _OUT_DIM:], g[:, :_OUT_DIM])
    bits = jnp.where((sel & 1) != 0, w & jnp.uint32(0xFFFF0000), w << 16)
    o_ref[...] = lax.bitcast_convert_type(bits, jnp.float32)


def _select(gathered, par):
    return pl.pallas_call(
        _select_body,
        grid=(_BATCH // _SEL_ROWS,),
        in_specs=[
            pl.BlockSpec((_SEL_ROWS, _PACK), lambda i: (i, 0)),
            pl.BlockSpec((_SEL_ROWS,), lambda i: (i,)),
        ],
        out_specs=pl.BlockSpec((_SEL_ROWS, _OUT_DIM), lambda i: (i, 0)),
        out_shape=jax.ShapeDtypeStruct((_BATCH, _OUT_DIM), jnp.float32),
    )(gathered, par)


def kernel(index, weights):
    table = _transpose_packed(weights)
    idx = index.reshape(-1).astype(jnp.int32)
    h = (idx >= _HALF).astype(jnp.int32)
    r = idx - h * _HALF
    idx2 = (r >> 1).reshape(_BATCH // 128, 128)
    sel = h * 2 + (r & 1)
    gathered = _make_gather()(table, idx2)
    return _select(gathered, sel)
